# baseline (speedup 1.0000x reference)
import functools

import jax
import jax.numpy as jnp
import numpy as np
from jax import lax
from jax.experimental import pallas as pl
from jax.experimental.pallas import tpu as pltpu
from jax.experimental.pallas import tpu_sc as plsc

_Q = 128
_M = 64

_cheb_m = np.arange(_M)
_cheb_ang = np.pi * (2 * _cheb_m + 1) / (2 * _M)
_T_NODES = (50.0 + 50.0 * np.cos(_cheb_ang)).astype(np.float32).reshape(_M, 1)
_S_COEF = ((2.0 / _M) * np.cos(np.outer(_cheb_m, _cheb_ang))).astype(np.float32)
_S_COEF[0] *= 0.5


@functools.lru_cache(maxsize=None)
def _make_gather(N: int, D: int, E: int):
    info = plsc.get_sparse_core_info()
    NC, NS = info.num_cores, info.num_subcores
    NW = NC * NS
    assert E % _Q == 0
    mesh = plsc.VectorSubcoreMesh(core_axis_name="c", subcore_axis_name="s")

    rows_per_w = (E // NW) // 16 * 16
    leftover = E - NW * rows_per_w
    n_full = rows_per_w // _Q
    rem = rows_per_w - n_full * _Q
    assert E % 16 == 0 and rem % 16 == 0 and leftover % 16 == 0
    assert leftover <= _Q

    @functools.partial(
        pl.kernel,
        mesh=mesh,
        out_type=jax.ShapeDtypeStruct((E, D), jnp.int32),
        scratch_types=[
            pltpu.VMEM((rows_per_w,), jnp.int32),
            pltpu.VMEM((_Q, D), jnp.int32),
            pltpu.VMEM((_Q, D), jnp.int32),
            pltpu.SemaphoreType.DMA,
            pltpu.SemaphoreType.DMA,
        ],
    )
    def gather(table_hbm, idx_hbm, out_hbm, idx_v, rows_a, rows_b, sem_a, sem_b):
        wid = lax.axis_index("s") * NC + lax.axis_index("c")
        base = pl.multiple_of(wid * rows_per_w, 16)
        pltpu.sync_copy(idx_hbm.at[pl.ds(base, rows_per_w)], idx_v)
        bufs = (rows_a, rows_b)
        sems = (sem_a, sem_b)

        def fire(j, b):
            pltpu.async_copy(table_hbm.at[idx_v.at[pl.ds(j * _Q, _Q)]],
                             bufs[b], sems[b])

        def drain(j, b):
            pltpu.make_async_copy(table_hbm.at[idx_v.at[pl.ds(0, _Q)]],
                                  bufs[b], sems[b]).wait()
            pltpu.sync_copy(
                bufs[b],
                out_hbm.at[pl.ds(pl.multiple_of(base + j * _Q, 16), _Q)])

        fire(0, 0)

        @pl.loop(0, n_full, step=2)
        def _ring(g):
            @pl.when(g + 1 < n_full)
            def _():
                fire(g + 1, 1)

            drain(g, 0)

            @pl.when(g + 2 < n_full)
            def _():
                fire(g + 2, 0)

            @pl.when(g + 1 < n_full)
            def _():
                drain(g + 1, 1)

        if rem:
            r0 = n_full * _Q
            pltpu.async_copy(table_hbm.at[idx_v.at[pl.ds(r0, rem)]],
                             rows_a.at[pl.ds(0, rem)], sem_a).wait()
            pltpu.sync_copy(
                rows_a.at[pl.ds(0, rem)],
                out_hbm.at[pl.ds(pl.multiple_of(base + r0, 16), rem)])

        if leftover:
            @pl.when(wid == 0)
            def _tail():
                lbase = NW * rows_per_w
                pltpu.sync_copy(idx_hbm.at[pl.ds(lbase, leftover)],
                                idx_v.at[pl.ds(0, leftover)])
                pltpu.async_copy(table_hbm.at[idx_v.at[pl.ds(0, leftover)]],
                                 rows_b.at[pl.ds(0, leftover)], sem_b).wait()
                pltpu.sync_copy(rows_b.at[pl.ds(0, leftover)],
                                out_hbm.at[pl.ds(lbase, leftover)])

    return gather


@functools.lru_cache(maxsize=None)
def _make_scatter(N: int, D: int, E: int):
    info = plsc.get_sparse_core_info()
    NC, NS = info.num_cores, info.num_subcores
    DH = D // NC
    NP = 2
    RH = N // NP
    AR = 5120
    ZQ = 40
    ZC = AR // NS // ZQ
    WQ = 200
    n_wb = RH // WQ
    edges_per_sub = E // NS
    n_full = edges_per_sub // _Q
    n4 = (n_full // 4) * 4
    rem = edges_per_sub - n_full * _Q
    rem_pad = ((rem + 15) // 16) * 16
    assert E % NS == 0 and edges_per_sub % 8 == 0 and rem % 8 == 0
    assert n_full >= 4
    mesh = plsc.VectorSubcoreMesh(core_axis_name="c", subcore_axis_name="s")

    @functools.partial(
        pl.kernel,
        mesh=mesh,
        out_type=jax.ShapeDtypeStruct((N, D), jnp.float32),
        scratch_types=[
            pltpu.VMEM((n_full * _Q + rem_pad,), jnp.int32),
            pltpu.VMEM((_Q,), jnp.int32),
            pltpu.VMEM((_Q,), jnp.int32),
            pltpu.VMEM((_Q,), jnp.int32),
            pltpu.VMEM((_Q,), jnp.int32),
            pltpu.VMEM((max(rem_pad, 16),), jnp.int32),
            pltpu.VMEM((_Q, DH), jnp.float32),
            pltpu.VMEM((_Q, DH), jnp.float32),
            pltpu.VMEM((_Q, DH), jnp.float32),
            pltpu.VMEM((_Q, DH), jnp.float32),
            pltpu.VMEM((ZQ, DH), jnp.float32),
            pltpu.VMEM_SHARED((AR, DH), jnp.float32),
            pltpu.SemaphoreType.DMA,
            pltpu.SemaphoreType.DMA,
            pltpu.SemaphoreType.DMA,
            pltpu.SemaphoreType.DMA,
            pltpu.SemaphoreType.DMA,
            pltpu.SemaphoreType.DMA,
            pltpu.SemaphoreType.DMA,
            pltpu.SemaphoreType.DMA,
        ],
    )
    def scatter(msg_hbm, dst_hbm, out_hbm, idx_v, adj_a, adj_b, adj_c, adj_d,
                adj_r, rows_a, rows_b, rows_c, rows_d, zero_v, acc,
                lsem_a, lsem_b, lsem_c, lsem_d,
                asem_a, asem_b, asem_c, asem_d):
        c = lax.axis_index("c")
        s = lax.axis_index("s")
        col0 = c * DH
        ebase = s * edges_per_sub
        bufs = (rows_a, rows_b, rows_c, rows_d)
        adjs = (adj_a, adj_b, adj_c, adj_d)
        lsems = (lsem_a, lsem_b, lsem_c, lsem_d)
        asems = (asem_a, asem_b, asem_c, asem_d)

        pltpu.sync_copy(dst_hbm.at[pl.ds(ebase, edges_per_sub)],
                        idx_v.at[pl.ds(0, edges_per_sub)])

        @pl.loop(0, ZQ)
        def _z(i):
            for j in range(DH // 16):
                zero_v[i, pl.ds(j * 16, 16)] = jnp.zeros((16,), jnp.float32)

        def load_fire(j, sl):
            pltpu.async_copy(
                msg_hbm.at[pl.ds(ebase + j * _Q, _Q), pl.ds(col0, DH)],
                bufs[sl], lsems[sl])

        def load_wait(sl):
            pltpu.make_async_copy(
                msg_hbm.at[pl.ds(ebase, _Q), pl.ds(col0, DH)],
                bufs[sl], lsems[sl]).wait()

        def add_fire(sl):
            pltpu.async_copy(bufs[sl], acc.at[adjs[sl]], asems[sl], add=True)

        def add_wait(sl):
            pltpu.make_async_copy(bufs[sl], acc.at[adjs[sl]], asems[sl]).wait()

        def adjust(j, sl, row0):
            for jj in range(_Q // 16):
                v = idx_v[pl.ds(j * _Q + jj * 16, 16)] - row0
                ok = (v >= 0) & (v < RH)
                adjs[sl][pl.ds(jj * 16, 16)] = jnp.where(
                    ok, v, jnp.full((16,), RH, jnp.int32))

        def step(j, k, row0, first_round):
            nxt = (k + 2) % 4
            if first_round:
                if j >= 2:
                    add_wait(nxt)
            else:
                add_wait(nxt)

            @pl.when(j + 2 < n_full)
            def _():
                load_fire(j + 2, nxt)

            load_wait(k)
            adjust(j, k, row0)
            add_fire(k)

        for p in range(NP):
            row0 = p * RH
            @pl.loop(0, ZC)
            def _zf(zi):
                pltpu.sync_copy(zero_v,
                                acc.at[pl.ds((s * ZC + zi) * ZQ, ZQ)])

            plsc.subcore_barrier()

            load_fire(0, 0)
            load_fire(1, 1)
            for k in range(4):
                step(k, k, row0, True)

            @pl.loop(4, n4, step=4)
            def _ring(j0):
                for k in range(4):
                    step(j0 + k, k, row0, False)

            for j in range(n4, n_full):
                step(j, j % 4, row0, False)

            add_wait((n_full - 2) % 4)
            add_wait((n_full - 1) % 4)

            if rem:
                r0 = n_full * _Q
                pltpu.async_copy(
                    msg_hbm.at[pl.ds(ebase + r0, rem), pl.ds(col0, DH)],
                    rows_a.at[pl.ds(0, rem)], lsem_a).wait()
                for jj in range(rem_pad // 16):
                    v = idx_v[pl.ds(r0 + jj * 16, 16)] - row0
                    ok = (v >= 0) & (v < RH)
                    if (jj + 1) * 16 > rem:
                        lane = lax.iota(jnp.int32, 16)
                        ok = ok & (lane < (rem - jj * 16))
                    adj_r[pl.ds(jj * 16, 16)] = jnp.where(
                        ok, v, jnp.full((16,), RH, jnp.int32))
                pltpu.sync_copy(rows_a.at[pl.ds(0, rem_pad)],
                                acc.at[adj_r], add=True)

            plsc.subcore_barrier()

            @pl.loop(0, (n_wb - s + NS - 1) // NS)
            def _wb(j):
                r0 = (s + j * NS) * WQ
                pltpu.sync_copy(acc.at[pl.ds(r0, WQ)],
                                out_hbm.at[pl.ds(row0 + r0, WQ),
                                           pl.ds(col0, DH)])

            if p != NP - 1:
                plsc.subcore_barrier()

    return scatter


def _prep_body(x_ref, w_ref, b_ref, Wm_ref, tn_ref, S_ref, y_ref, C_ref):
    y_ref[...] = jnp.dot(x_ref[...], Wm_ref[...],
                         preferred_element_type=jnp.float32
                         ).astype(jnp.bfloat16)

    @pl.when(pl.program_id(0) == 0)
    def _():
        G = jnp.dot(jnp.cos(tn_ref[...] * w_ref[...] + b_ref[...]),
                    Wm_ref[...], preferred_element_type=jnp.float32)
        C_ref[...] = jnp.dot(S_ref[...], G, preferred_element_type=jnp.float32)


def _msg_body(ysrc_ref, t3_ref, C_ref, out_ref):
    SB = t3_ref.shape[2]
    th2 = (t3_ref[0] - 50.0) * 0.02
    cols = [jnp.ones((8, SB), jnp.float32), th2]
    for _ in range(2, _M):
        cols.append(2.0 * th2 * cols[-1] - cols[-2])
    P3 = jnp.stack(cols, axis=0)
    C = C_ref[...]
    v = ysrc_ref[...]
    lo = lax.bitcast_convert_type(v << 16, jnp.float32)
    hi = lax.bitcast_convert_type(v & jnp.int32(-65536), jnp.float32)
    ys = jnp.concatenate([lo, hi], axis=1)
    for u in range(8):
        tm = lax.dot_general(P3[:, u, :], C, (((0,), (0,)), ((), ())),
                             preferred_element_type=jnp.float32)
        out_ref[pl.ds(u * SB, SB), :] = jnp.maximum(
            ys[u * SB:(u + 1) * SB, :] + tm, 0.0)


def _tail_body(x_ref, aggA_ref, aggB_ref, Wt_ref, Wb_ref, W1_ref, b1_ref,
               W2_ref, b2_ref, out_ref):
    agg = aggA_ref[...] + aggB_ref[...]
    h = jnp.maximum(
        jnp.dot(x_ref[...], Wt_ref[...], preferred_element_type=jnp.float32)
        + jnp.dot(agg, Wb_ref[...], preferred_element_type=jnp.float32),
        0.0)
    hid = jnp.maximum(
        jnp.dot(h, W1_ref[...], preferred_element_type=jnp.float32)
        + b1_ref[...], 0.0)
    logits = jnp.dot(hid, W2_ref[...], preferred_element_type=jnp.float32) \
        + b2_ref[...]
    m = jnp.max(logits, axis=-1, keepdims=True)
    e = jnp.exp(logits - m)
    out_ref[...] = e / jnp.sum(e, axis=-1, keepdims=True)


def kernel(x, edge_index, edge_times, time_w, time_b, W_msg, W_upd,
           W1, b1, W2, b2):
    N, D = x.shape
    E = edge_times.shape[0]
    K = W2.shape[1]
    KP = 8

    src = edge_index[0]
    dst = edge_index[1]

    BN = 2000
    y, C = pl.pallas_call(
        _prep_body,
        grid=(N // BN,),
        in_specs=[
            pl.BlockSpec((BN, D), lambda i: (i, 0)),
            pl.BlockSpec((1, D), lambda i: (0, 0)),
            pl.BlockSpec((1, D), lambda i: (0, 0)),
            pl.BlockSpec((D, D), lambda i: (0, 0)),
            pl.BlockSpec((_M, 1), lambda i: (0, 0)),
            pl.BlockSpec((_M, _M), lambda i: (0, 0)),
        ],
        out_specs=[
            pl.BlockSpec((BN, D), lambda i: (i, 0)),
            pl.BlockSpec((_M, D), lambda i: (0, 0)),
        ],
        out_shape=[
            jax.ShapeDtypeStruct((N, D), jnp.bfloat16),
            jax.ShapeDtypeStruct((_M, D), jnp.float32),
        ],
    )(x, time_w[None, :], time_b[None, :], W_msg,
      jnp.asarray(_T_NODES), jnp.asarray(_S_COEF))

    BE = 3200
    SB = BE // 8
    EH = E // 2
    nbh = EH // BE
    t3 = edge_times.reshape(E // BE, 8, SB)

    def msg_half(ysrc_h, off):
        return pl.pallas_call(
            _msg_body,
            grid=(nbh,),
            in_specs=[
                pl.BlockSpec((BE, D // 2), lambda i: (i, 0)),
                pl.BlockSpec((1, 8, SB), lambda i, o=off: (i + o, 0, 0)),
                pl.BlockSpec((_M, D), lambda i: (0, 0)),
            ],
            out_specs=pl.BlockSpec((BE, D), lambda i: (i, 0)),
            out_shape=jax.ShapeDtypeStruct((EH, D), jnp.float32),
        )(ysrc_h, t3, C)

    y32 = lax.bitcast_convert_type(
        jnp.stack([y[:, :D // 2], y[:, D // 2:]], axis=-1), jnp.int32)

    gat = _make_gather(N, D // 2, EH)
    scat = _make_scatter(N, D, EH)
    ysrcA = gat(y32, src[:EH])
    msgA = msg_half(ysrcA, 0)
    ysrcB = gat(y32, src[EH:])
    msgB = msg_half(ysrcB, nbh)
    aggA = scat(msgA, dst[:EH])
    aggB = scat(msgB, dst[EH:])

    W2p = jnp.zeros((D, KP), jnp.float32).at[:, :K].set(W2)
    b2p = jnp.full((KP,), -1e30, jnp.float32).at[:K].set(b2)

    BT = 2000
    pi_pad = pl.pallas_call(
        _tail_body,
        grid=(N // BT,),
        in_specs=[
            pl.BlockSpec((BT, D), lambda i: (i, 0)),
            pl.BlockSpec((BT, D), lambda i: (i, 0)),
            pl.BlockSpec((BT, D), lambda i: (i, 0)),
            pl.BlockSpec((D, D), lambda i: (0, 0)),
            pl.BlockSpec((D, D), lambda i: (0, 0)),
            pl.BlockSpec((D, D), lambda i: (0, 0)),
            pl.BlockSpec((1, D), lambda i: (0, 0)),
            pl.BlockSpec((D, KP), lambda i: (0, 0)),
            pl.BlockSpec((1, KP), lambda i: (0, 0)),
        ],
        out_specs=pl.BlockSpec((BT, KP), lambda i: (i, 0)),
        out_shape=jax.ShapeDtypeStruct((N, KP), jnp.float32),
    )(x, aggA, aggB, W_upd[:D], W_upd[D:], W1, b1[None, :], W2p, b2p[None, :])

    return pi_pad[:, :K]


# speedup vs baseline: 1.0145x; 1.0145x over previous
import functools

import jax
import jax.numpy as jnp
import numpy as np
from jax import lax
from jax.experimental import pallas as pl
from jax.experimental.pallas import tpu as pltpu
from jax.experimental.pallas import tpu_sc as plsc

_Q = 128
_M = 64

_cheb_m = np.arange(_M)
_cheb_ang = np.pi * (2 * _cheb_m + 1) / (2 * _M)
_T_NODES = (50.0 + 50.0 * np.cos(_cheb_ang)).astype(np.float32).reshape(_M, 1)
_S_COEF = ((2.0 / _M) * np.cos(np.outer(_cheb_m, _cheb_ang))).astype(np.float32)
_S_COEF[0] *= 0.5


@functools.lru_cache(maxsize=None)
def _make_gather(N: int, D: int, E: int):
    info = plsc.get_sparse_core_info()
    NC, NS = info.num_cores, info.num_subcores
    NW = NC * NS
    assert E % _Q == 0
    mesh = plsc.VectorSubcoreMesh(core_axis_name="c", subcore_axis_name="s")

    rows_per_w = (E // NW) // 16 * 16
    leftover = E - NW * rows_per_w
    n_full = rows_per_w // _Q
    rem = rows_per_w - n_full * _Q
    assert E % 16 == 0 and rem % 16 == 0 and leftover % 16 == 0
    assert leftover <= _Q

    @functools.partial(
        pl.kernel,
        mesh=mesh,
        out_type=jax.ShapeDtypeStruct((E, D), jnp.int32),
        scratch_types=[
            pltpu.VMEM((rows_per_w,), jnp.int32),
            pltpu.VMEM((_Q, D), jnp.int32),
            pltpu.VMEM((_Q, D), jnp.int32),
            pltpu.SemaphoreType.DMA,
            pltpu.SemaphoreType.DMA,
        ],
    )
    def gather(table_hbm, idx_hbm, out_hbm, idx_v, rows_a, rows_b, sem_a, sem_b):
        wid = lax.axis_index("s") * NC + lax.axis_index("c")
        base = pl.multiple_of(wid * rows_per_w, 16)
        pltpu.sync_copy(idx_hbm.at[pl.ds(base, rows_per_w)], idx_v)
        bufs = (rows_a, rows_b)
        sems = (sem_a, sem_b)

        def fire(j, b):
            pltpu.async_copy(table_hbm.at[idx_v.at[pl.ds(j * _Q, _Q)]],
                             bufs[b], sems[b])

        def drain(j, b):
            pltpu.make_async_copy(table_hbm.at[idx_v.at[pl.ds(0, _Q)]],
                                  bufs[b], sems[b]).wait()
            pltpu.sync_copy(
                bufs[b],
                out_hbm.at[pl.ds(pl.multiple_of(base + j * _Q, 16), _Q)])

        fire(0, 0)

        @pl.loop(0, n_full, step=2)
        def _ring(g):
            @pl.when(g + 1 < n_full)
            def _():
                fire(g + 1, 1)

            drain(g, 0)

            @pl.when(g + 2 < n_full)
            def _():
                fire(g + 2, 0)

            @pl.when(g + 1 < n_full)
            def _():
                drain(g + 1, 1)

        if rem:
            r0 = n_full * _Q
            pltpu.async_copy(table_hbm.at[idx_v.at[pl.ds(r0, rem)]],
                             rows_a.at[pl.ds(0, rem)], sem_a).wait()
            pltpu.sync_copy(
                rows_a.at[pl.ds(0, rem)],
                out_hbm.at[pl.ds(pl.multiple_of(base + r0, 16), rem)])

        if leftover:
            @pl.when(wid == 0)
            def _tail():
                lbase = NW * rows_per_w
                pltpu.sync_copy(idx_hbm.at[pl.ds(lbase, leftover)],
                                idx_v.at[pl.ds(0, leftover)])
                pltpu.async_copy(table_hbm.at[idx_v.at[pl.ds(0, leftover)]],
                                 rows_b.at[pl.ds(0, leftover)], sem_b).wait()
                pltpu.sync_copy(rows_b.at[pl.ds(0, leftover)],
                                out_hbm.at[pl.ds(lbase, leftover)])

    return gather


@functools.lru_cache(maxsize=None)
def _make_scatter(N: int, D: int, E: int):
    info = plsc.get_sparse_core_info()
    NC, NS = info.num_cores, info.num_subcores
    DH = D // NC
    NP = 2
    RH = N // NP
    AR = 5120
    ZQ = AR // NS
    WQ = 200
    n_wb = RH // WQ
    edges_per_sub = E // NS
    n_full = edges_per_sub // _Q
    rem = edges_per_sub - n_full * _Q
    rem_pad = ((rem + 15) // 16) * 16
    assert E % NS == 0 and edges_per_sub % 8 == 0 and rem % 8 == 0
    mesh = plsc.VectorSubcoreMesh(core_axis_name="c", subcore_axis_name="s")

    @functools.partial(
        pl.kernel,
        mesh=mesh,
        out_type=jax.ShapeDtypeStruct((N, D), jnp.float32),
        scratch_types=[
            pltpu.VMEM((n_full * _Q + rem_pad,), jnp.int32),
            pltpu.VMEM((_Q,), jnp.int32),
            pltpu.VMEM((_Q,), jnp.int32),
            pltpu.VMEM((max(rem_pad, 16),), jnp.int32),
            pltpu.VMEM((_Q, DH), jnp.float32),
            pltpu.VMEM((_Q, DH), jnp.float32),
            pltpu.VMEM((ZQ, DH), jnp.float32),
            pltpu.VMEM_SHARED((AR, DH), jnp.float32),
            pltpu.SemaphoreType.DMA,
            pltpu.SemaphoreType.DMA,
        ],
    )
    def scatter(msg_hbm, dst_hbm, out_hbm, idx_v, adj_a, adj_b, adj_r,
                rows_a, rows_b, zero_v, acc, sem_a, sem_b):
        c = lax.axis_index("c")
        s = lax.axis_index("s")
        col0 = c * DH
        ebase = s * edges_per_sub
        bufs = (rows_a, rows_b)
        adjs = (adj_a, adj_b)
        sems = (sem_a, sem_b)

        pltpu.sync_copy(dst_hbm.at[pl.ds(ebase, edges_per_sub)],
                        idx_v.at[pl.ds(0, edges_per_sub)])

        @pl.loop(0, ZQ)
        def _z(i):
            for j in range(DH // 16):
                zero_v[i, pl.ds(j * 16, 16)] = jnp.zeros((16,), jnp.float32)

        def fire(j, b):
            pltpu.async_copy(
                msg_hbm.at[pl.ds(ebase + j * _Q, _Q), pl.ds(col0, DH)],
                bufs[b], sems[b])

        def adjust(j, b, row0):
            for jj in range(_Q // 16):
                v = idx_v[pl.ds(j * _Q + jj * 16, 16)] - row0
                ok = (v >= 0) & (v < RH)
                adjs[b][pl.ds(jj * 16, 16)] = jnp.where(
                    ok, v, jnp.full((16,), RH, jnp.int32))

        def drain_add(j, b):
            pltpu.make_async_copy(
                msg_hbm.at[pl.ds(ebase, _Q), pl.ds(col0, DH)],
                bufs[b], sems[b]).wait()
            pltpu.sync_copy(bufs[b], acc.at[adjs[b]], add=True)

        for p in range(NP):
            row0 = p * RH
            pltpu.sync_copy(zero_v, acc.at[pl.ds(s * ZQ, ZQ)])
            plsc.subcore_barrier()

            fire(0, 0)

            @pl.loop(0, n_full, step=2)
            def _ring(g):
                @pl.when(g + 1 < n_full)
                def _():
                    fire(g + 1, 1)

                adjust(g, 0, row0)
                drain_add(g, 0)

                @pl.when(g + 2 < n_full)
                def _():
                    fire(g + 2, 0)

                @pl.when(g + 1 < n_full)
                def _():
                    adjust(g + 1, 1, row0)
                    drain_add(g + 1, 1)

            if rem:
                r0 = n_full * _Q
                pltpu.async_copy(
                    msg_hbm.at[pl.ds(ebase + r0, rem), pl.ds(col0, DH)],
                    rows_a.at[pl.ds(0, rem)], sem_a).wait()
                for jj in range(rem_pad // 16):
                    v = idx_v[pl.ds(r0 + jj * 16, 16)] - row0
                    ok = (v >= 0) & (v < RH)
                    if (jj + 1) * 16 > rem:
                        lane = lax.iota(jnp.int32, 16)
                        ok = ok & (lane < (rem - jj * 16))
                    adj_r[pl.ds(jj * 16, 16)] = jnp.where(
                        ok, v, jnp.full((16,), RH, jnp.int32))
                pltpu.sync_copy(rows_a.at[pl.ds(0, rem_pad)],
                                acc.at[adj_r], add=True)

            plsc.subcore_barrier()

            @pl.loop(0, (n_wb - s + NS - 1) // NS)
            def _wb(j):
                r0 = (s + j * NS) * WQ
                pltpu.sync_copy(acc.at[pl.ds(r0, WQ)],
                                out_hbm.at[pl.ds(row0 + r0, WQ),
                                           pl.ds(col0, DH)])

            if p != NP - 1:
                plsc.subcore_barrier()

    return scatter


def _prep_body(x_ref, w_ref, b_ref, Wm_ref, tn_ref, S_ref, y_ref, C_ref):
    y_ref[...] = jnp.dot(x_ref[...], Wm_ref[...],
                         preferred_element_type=jnp.float32
                         ).astype(jnp.bfloat16)

    @pl.when(pl.program_id(0) == 0)
    def _():
        G = jnp.dot(jnp.cos(tn_ref[...] * w_ref[...] + b_ref[...]),
                    Wm_ref[...], preferred_element_type=jnp.float32)
        C_ref[...] = jnp.dot(S_ref[...], G, preferred_element_type=jnp.float32)


def _msg_body(ysrc_ref, t3_ref, C_ref, out_ref):
    SB = t3_ref.shape[2]
    th2 = (t3_ref[0] - 50.0) * 0.02
    cols = [jnp.ones((8, SB), jnp.float32), th2]
    for _ in range(2, _M):
        cols.append(2.0 * th2 * cols[-1] - cols[-2])
    P3 = jnp.stack(cols, axis=0)
    C = C_ref[...]
    v = ysrc_ref[...]
    lo = lax.bitcast_convert_type(v << 16, jnp.float32)
    hi = lax.bitcast_convert_type(v & jnp.int32(-65536), jnp.float32)
    ys = jnp.concatenate([lo, hi], axis=1)
    for u in range(8):
        tm = lax.dot_general(P3[:, u, :], C, (((0,), (0,)), ((), ())),
                             preferred_element_type=jnp.float32)
        out_ref[pl.ds(u * SB, SB), :] = jnp.maximum(
            ys[u * SB:(u + 1) * SB, :] + tm, 0.0)


def _tail_body(x_ref, aggA_ref, aggB_ref, Wt_ref, Wb_ref, W1_ref, b1_ref,
               W2_ref, b2_ref, out_ref):
    agg = aggA_ref[...] + aggB_ref[...]
    h = jnp.maximum(
        jnp.dot(x_ref[...], Wt_ref[...], preferred_element_type=jnp.float32)
        + jnp.dot(agg, Wb_ref[...], preferred_element_type=jnp.float32),
        0.0)
    hid = jnp.maximum(
        jnp.dot(h, W1_ref[...], preferred_element_type=jnp.float32)
        + b1_ref[...], 0.0)
    logits = jnp.dot(hid, W2_ref[...], preferred_element_type=jnp.float32) \
        + b2_ref[...]
    m = jnp.max(logits, axis=-1, keepdims=True)
    e = jnp.exp(logits - m)
    out_ref[...] = e / jnp.sum(e, axis=-1, keepdims=True)


def kernel(x, edge_index, edge_times, time_w, time_b, W_msg, W_upd,
           W1, b1, W2, b2):
    N, D = x.shape
    E = edge_times.shape[0]
    K = W2.shape[1]
    KP = 8

    src = edge_index[0]
    dst = edge_index[1]

    BN = 2000
    y, C = pl.pallas_call(
        _prep_body,
        grid=(N // BN,),
        in_specs=[
            pl.BlockSpec((BN, D), lambda i: (i, 0)),
            pl.BlockSpec((1, D), lambda i: (0, 0)),
            pl.BlockSpec((1, D), lambda i: (0, 0)),
            pl.BlockSpec((D, D), lambda i: (0, 0)),
            pl.BlockSpec((_M, 1), lambda i: (0, 0)),
            pl.BlockSpec((_M, _M), lambda i: (0, 0)),
        ],
        out_specs=[
            pl.BlockSpec((BN, D), lambda i: (i, 0)),
            pl.BlockSpec((_M, D), lambda i: (0, 0)),
        ],
        out_shape=[
            jax.ShapeDtypeStruct((N, D), jnp.bfloat16),
            jax.ShapeDtypeStruct((_M, D), jnp.float32),
        ],
    )(x, time_w[None, :], time_b[None, :], W_msg,
      jnp.asarray(_T_NODES), jnp.asarray(_S_COEF))

    BE = 3200
    SB = BE // 8
    EH = E // 2
    nbh = EH // BE
    t3 = edge_times.reshape(E // BE, 8, SB)

    def msg_half(ysrc_h, off):
        return pl.pallas_call(
            _msg_body,
            grid=(nbh,),
            in_specs=[
                pl.BlockSpec((BE, D // 2), lambda i: (i, 0)),
                pl.BlockSpec((1, 8, SB), lambda i, o=off: (i + o, 0, 0)),
                pl.BlockSpec((_M, D), lambda i: (0, 0)),
            ],
            out_specs=pl.BlockSpec((BE, D), lambda i: (i, 0)),
            out_shape=jax.ShapeDtypeStruct((EH, D), jnp.float32),
        )(ysrc_h, t3, C)

    y32 = lax.bitcast_convert_type(
        jnp.stack([y[:, :D // 2], y[:, D // 2:]], axis=-1), jnp.int32)

    gat = _make_gather(N, D // 2, EH)
    scat = _make_scatter(N, D, EH)
    ysrcA = gat(y32, src[:EH])
    msgA = msg_half(ysrcA, 0)
    ysrcB = gat(y32, src[EH:])
    msgB = msg_half(ysrcB, nbh)
    aggA = scat(msgA, dst[:EH])
    aggB = scat(msgB, dst[EH:])

    W2p = jnp.zeros((D, KP), jnp.float32).at[:, :K].set(W2)
    b2p = jnp.full((KP,), -1e30, jnp.float32).at[:K].set(b2)

    BT = 2000
    pi_pad = pl.pallas_call(
        _tail_body,
        grid=(N // BT,),
        in_specs=[
            pl.BlockSpec((BT, D), lambda i: (i, 0)),
            pl.BlockSpec((BT, D), lambda i: (i, 0)),
            pl.BlockSpec((BT, D), lambda i: (i, 0)),
            pl.BlockSpec((D, D), lambda i: (0, 0)),
            pl.BlockSpec((D, D), lambda i: (0, 0)),
            pl.BlockSpec((D, D), lambda i: (0, 0)),
            pl.BlockSpec((1, D), lambda i: (0, 0)),
            pl.BlockSpec((D, KP), lambda i: (0, 0)),
            pl.BlockSpec((1, KP), lambda i: (0, 0)),
        ],
        out_specs=pl.BlockSpec((BT, KP), lambda i: (i, 0)),
        out_shape=jax.ShapeDtypeStruct((N, KP), jnp.float32),
    )(x, aggA, aggB, W_upd[:D], W_upd[D:], W1, b1[None, :], W2p, b2p[None, :])

    return pi_pad[:, :K]
